# baseline (speedup 1.0000x reference)
import functools

import jax
import jax.numpy as jnp
from jax import lax
from jax.experimental import pallas as pl
from jax.experimental.pallas import tpu as pltpu
from jax.experimental.pallas import tpu_sc as plsc

D_MODEL = 32
SPLIT = 256


@functools.partial(jax.jit, static_argnames=("b", "chunk"))
def _embed_sc(idx_flat, a_tab, b_tab, b, chunk):
    info = plsc.get_sparse_core_info()
    nw = info.num_cores * info.num_subcores
    b_per_w = b // nw
    n_chunks = b_per_w // chunk
    assert n_chunks % 2 == 0 and chunk % 16 == 0
    n_groups = chunk // 16
    a_n = a_tab.shape[0]
    b_n = b_tab.shape[0]
    mesh = plsc.VectorSubcoreMesh(core_axis_name="c", subcore_axis_name="s")

    @functools.partial(
        pl.kernel,
        mesh=mesh,
        out_type=jax.ShapeDtypeStruct((b * D_MODEL,), jnp.float32),
        scratch_types=[
            pltpu.VMEM((a_n,), jnp.float32),
            pltpu.VMEM((b_n,), jnp.float32),
            pltpu.VMEM((chunk,), jnp.int32),
            pltpu.VMEM((chunk,), jnp.int32),
            pltpu.VMEM((chunk * D_MODEL,), jnp.float32),
            pltpu.VMEM((chunk * D_MODEL,), jnp.float32),
            pltpu.SemaphoreType.DMA,
            pltpu.SemaphoreType.DMA,
            pltpu.SemaphoreType.DMA,
            pltpu.SemaphoreType.DMA,
            pltpu.SemaphoreType.DMA,
        ],
        compiler_params=pltpu.CompilerParams(
            use_tc_tiling_on_sc=False, needs_layout_passes=False),
    )
    def k(idx_hbm, a_hbm, b_hbm, out_hbm, a_v, b_v, idx0, idx1, rows0, rows1,
          tsem, isem0, isem1, osem0, osem1):
        wid = lax.axis_index("s") * info.num_cores + lax.axis_index("c")
        base = wid * b_per_w
        idx_v = (idx0, idx1)
        rows_v = (rows0, rows1)
        isem = (isem0, isem1)
        osem = (osem0, osem1)

        pltpu.async_copy(a_hbm, a_v, tsem)
        pltpu.async_copy(b_hbm, b_v, tsem)
        pltpu.async_copy(idx_hbm.at[pl.ds(base, chunk)], idx0, isem0)
        pltpu.async_copy(idx_hbm.at[pl.ds(base + chunk, chunk)], idx1, isem1)
        pltpu.make_async_copy(a_hbm, a_v, tsem).wait()
        pltpu.make_async_copy(b_hbm, b_v, tsem).wait()

        lane = lax.iota(jnp.int32, 16)

        def body(h, carry):
            for bi in range(2):
                g = 2 * h + bi
                start = base + g * chunk
                pltpu.make_async_copy(
                    idx_hbm.at[pl.ds(start, chunk)], idx_v[bi], isem[bi]).wait()

                @pl.when(g >= 2)
                def _():
                    pltpu.make_async_copy(
                        rows_v[bi],
                        out_hbm.at[pl.ds((start - 2 * chunk) * D_MODEL,
                                         chunk * D_MODEL)],
                        osem[bi]).wait()

                def compute(t, c):
                    xv = plsc.load_gather(idx_v[bi], [t * 16 + lane])
                    hi = lax.shift_right_logical(xv, 8)
                    lo = lax.bitwise_and(xv, 255)
                    base_a = hi * D_MODEL
                    base_b = lo * D_MODEL
                    addr0 = t * (16 * D_MODEL) + lane * D_MODEL
                    for m in range(D_MODEL // 2):
                        s_a = plsc.load_gather(a_v, [base_a + (2 * m)])
                        c_a = plsc.load_gather(a_v, [base_a + (2 * m + 1)])
                        s_b = plsc.load_gather(b_v, [base_b + (2 * m)])
                        c_b = plsc.load_gather(b_v, [base_b + (2 * m + 1)])
                        even = s_a * c_b + c_a * s_b
                        odd = c_a * c_b - s_a * s_b
                        plsc.store_scatter(rows_v[bi], [addr0 + (2 * m)], even)
                        plsc.store_scatter(rows_v[bi], [addr0 + (2 * m + 1)], odd)
                    return c

                lax.fori_loop(0, n_groups, compute, 0)

                @pl.when(g + 2 < n_chunks)
                def _():
                    pltpu.async_copy(
                        idx_hbm.at[pl.ds(start + 2 * chunk, chunk)],
                        idx_v[bi], isem[bi])

                pltpu.async_copy(
                    rows_v[bi],
                    out_hbm.at[pl.ds(start * D_MODEL, chunk * D_MODEL)],
                    osem[bi])
            return carry

        lax.fori_loop(0, n_chunks // 2, body, 0)

        last = base + (n_chunks - 2) * chunk
        pltpu.make_async_copy(
            rows0, out_hbm.at[pl.ds(last * D_MODEL, chunk * D_MODEL)],
            osem0).wait()
        pltpu.make_async_copy(
            rows1, out_hbm.at[pl.ds((last + chunk) * D_MODEL, chunk * D_MODEL)],
            osem1).wait()

    return k(idx_flat, a_tab, b_tab)


def kernel(x, table):
    s0, s1 = x.shape
    b = s0 * s1
    idx_flat = x.reshape(b).astype(jnp.int32)
    a_tab = table[::SPLIT].reshape(-1)
    b_tab = table[:SPLIT].reshape(-1)
    out = _embed_sc(idx_flat, a_tab, b_tab, b, 1600)
    return out.reshape(s0, s1, D_MODEL)


# speedup vs baseline: 2.6730x; 2.6730x over previous
import functools

import jax
import jax.numpy as jnp
from jax import lax
from jax.experimental import pallas as pl
from jax.experimental.pallas import tpu as pltpu
from jax.experimental.pallas import tpu_sc as plsc

D_MODEL = 32
HALF = D_MODEL // 2
SPLIT = 256


@functools.partial(jax.jit, static_argnames=("b", "chunk"))
def _embed_sc(idx_flat, tabs, b, chunk):
    info = plsc.get_sparse_core_info()
    nw = info.num_cores * info.num_subcores
    b_per_w = b // nw
    n_chunks = b_per_w // chunk
    assert n_chunks % 2 == 0 and chunk % 16 == 0
    n_groups = chunk // 16
    tab_ns = [t.shape[0] for t in tabs]
    mesh = plsc.VectorSubcoreMesh(core_axis_name="c", subcore_axis_name="s")

    @functools.partial(
        pl.kernel,
        mesh=mesh,
        out_type=jax.ShapeDtypeStruct((b * D_MODEL,), jnp.float32),
        scratch_types=[
            tuple(pltpu.VMEM((n,), jnp.float32) for n in tab_ns),
            pltpu.VMEM((chunk,), jnp.int32),
            pltpu.VMEM((chunk,), jnp.int32),
            pltpu.VMEM((chunk * D_MODEL,), jnp.float32),
            pltpu.VMEM((chunk * D_MODEL,), jnp.float32),
            pltpu.SemaphoreType.DMA,
            pltpu.SemaphoreType.DMA,
            pltpu.SemaphoreType.DMA,
            pltpu.SemaphoreType.DMA,
            pltpu.SemaphoreType.DMA,
        ],
        compiler_params=pltpu.CompilerParams(
            use_tc_tiling_on_sc=False, needs_layout_passes=False),
    )
    def k(idx_hbm, a1_h, a2_h, v1_h, v2_h, w1_h, w2_h, z1_h, z2_h, out_hbm,
          tab_v, idx0, idx1, rows0, rows1,
          tsem, isem0, isem1, osem0, osem1):
        wid = lax.axis_index("s") * info.num_cores + lax.axis_index("c")
        base = wid * b_per_w
        tab_h = (a1_h, a2_h, v1_h, v2_h, w1_h, w2_h, z1_h, z2_h)
        idx_v = (idx0, idx1)
        rows_v = (rows0, rows1)
        isem = (isem0, isem1)
        osem = (osem0, osem1)

        for th, tv in zip(tab_h, tab_v):
            pltpu.async_copy(th, tv, tsem)
        pltpu.async_copy(idx_hbm.at[pl.ds(base, chunk)], idx0, isem0)
        pltpu.async_copy(idx_hbm.at[pl.ds(base + chunk, chunk)], idx1, isem1)
        for th, tv in zip(tab_h, tab_v):
            pltpu.make_async_copy(th, tv, tsem).wait()
        a1_v, a2_v, v1_v, v2_v, w1_v, w2_v, z1_v, z2_v = tab_v

        lane = lax.iota(jnp.int32, 16)

        def body(h, carry):
            for bi in range(2):
                g = 2 * h + bi
                start = base + g * chunk
                pltpu.make_async_copy(
                    idx_hbm.at[pl.ds(start, chunk)], idx_v[bi], isem[bi]).wait()

                @pl.when(g >= 2)
                def _():
                    pltpu.make_async_copy(
                        rows_v[bi],
                        out_hbm.at[pl.ds((start - 2 * chunk) * D_MODEL,
                                         chunk * D_MODEL)],
                        osem[bi]).wait()

                @plsc.parallel_loop(0, n_groups, step=1, unroll=2)
                def _(t):
                    xv = plsc.load_gather(idx_v[bi], [t * 16 + lane])
                    base_a = lax.shift_right_logical(xv, 8) * HALF
                    base_b = lax.bitwise_and(xv, 255) * HALF
                    addr0 = t * (16 * D_MODEL) + lane
                    for r in range(16):
                        rsel = jnp.full((16,), r, jnp.int32)
                        ba = base_a.at[rsel].get(mode="promise_in_bounds") + lane
                        bb = base_b.at[rsel].get(mode="promise_in_bounds") + lane
                        u1 = plsc.load_gather(a1_v, [ba])
                        u2 = plsc.load_gather(a2_v, [ba])
                        v1 = plsc.load_gather(v1_v, [ba])
                        v2 = plsc.load_gather(v2_v, [ba])
                        w1 = plsc.load_gather(w1_v, [bb])
                        w2 = plsc.load_gather(w2_v, [bb])
                        z1 = plsc.load_gather(z1_v, [bb])
                        z2 = plsc.load_gather(z2_v, [bb])
                        o1 = u1 * w1 + v1 * z1
                        o2 = u2 * w2 + v2 * z2
                        dst = addr0 + r * D_MODEL
                        plsc.store_scatter(rows_v[bi], [dst], o1)
                        plsc.store_scatter(rows_v[bi], [dst + 16], o2)

                @pl.when(g + 2 < n_chunks)
                def _():
                    pltpu.async_copy(
                        idx_hbm.at[pl.ds(start + 2 * chunk, chunk)],
                        idx_v[bi], isem[bi])

                pltpu.async_copy(
                    rows_v[bi],
                    out_hbm.at[pl.ds(start * D_MODEL, chunk * D_MODEL)],
                    osem[bi])
            return carry

        lax.fori_loop(0, n_chunks // 2, body, 0)

        last = base + (n_chunks - 2) * chunk
        pltpu.make_async_copy(
            rows0, out_hbm.at[pl.ds(last * D_MODEL, chunk * D_MODEL)],
            osem0).wait()
        pltpu.make_async_copy(
            rows1, out_hbm.at[pl.ds((last + chunk) * D_MODEL, chunk * D_MODEL)],
            osem1).wait()

    return k(idx_flat, *tabs)


def _derived_tables(table):
    a = table[::SPLIT]
    b_t = table[:SPLIT]
    c = jnp.arange(D_MODEL)
    v = a[:, c ^ 1]
    w = b_t[:, c | 1]
    sgn = jnp.where(c % 2 == 0, 1.0, -1.0).astype(table.dtype)
    z = b_t[:, c & ~1] * sgn
    out = []
    for t in (a, v, w, z):
        out.append(t[:, :HALF].reshape(-1))
        out.append(t[:, HALF:].reshape(-1))
    return tuple(out)


def kernel(x, table):
    s0, s1 = x.shape
    b = s0 * s1
    idx_flat = x.reshape(b).astype(jnp.int32)
    tabs = _derived_tables(table)
    out = _embed_sc(idx_flat, tabs, b, 1280)
    return out.reshape(s0, s1, D_MODEL)
